# baseline (speedup 1.0000x reference)
import functools

import jax
import jax.numpy as jnp
from jax import lax
from jax.experimental import pallas as pl
from jax.experimental.pallas import tpu as pltpu
from jax.experimental.pallas import tpu_sc as plsc

B = 1024
L = 200
H = 64
FLAT = B * L
NC = 2
NS = 16
NW = NC * NS
PER_W = FLAT // NW
IDX_MINOR = 100
IDX_ROWS = PER_W // IDX_MINOR
CHUNK = L
DMAS_PER_CHUNK = CHUNK // IDX_MINOR
NCHUNK = PER_W // CHUNK
LANES = 16


def _emb_body(x_hbm, tok_hbm, pos_hbm, out_hbm, idx_v, pos_v, rows_v, gsem):
    wid = lax.axis_index("s") * NC + lax.axis_index("c")
    pltpu.sync_copy(x_hbm.at[wid], idx_v)
    pltpu.sync_copy(pos_hbm.at[pl.ds(0, L)], pos_v)

    def chunk_body(c, carry):
        cps = []
        for h in range(DMAS_PER_CHUNK):
            cps.append(
                pltpu.async_copy(
                    tok_hbm.at[idx_v.at[c * DMAS_PER_CHUNK + h]],
                    rows_v.at[pl.ds(h * IDX_MINOR, IDX_MINOR)],
                    gsem,
                )
            )
        for cp in cps:
            cp.wait()

        def add_body(r, carry2):
            for k in range(H // LANES):
                s = pl.ds(k * LANES, LANES)
                rows_v[r, s] = rows_v[r, s] + pos_v[r, s]
            return carry2

        lax.fori_loop(0, CHUNK, add_body, 0)

        pltpu.sync_copy(
            rows_v,
            out_hbm.at[pl.ds(wid * PER_W + c * CHUNK, CHUNK)],
        )
        return carry

    lax.fori_loop(0, NCHUNK, chunk_body, 0)


@functools.cache
def _build_kernel():
    return functools.partial(
        pl.kernel,
        out_type=jax.ShapeDtypeStruct((FLAT, H), jnp.float32),
        mesh=plsc.VectorSubcoreMesh(core_axis_name="c", subcore_axis_name="s"),
        scratch_types=[
            pltpu.VMEM((IDX_ROWS, IDX_MINOR), jnp.int32),
            pltpu.VMEM((L, H), jnp.float32),
            pltpu.VMEM((CHUNK, H), jnp.float32),
            pltpu.SemaphoreType.DMA,
        ],
        compiler_params=pltpu.CompilerParams(use_tc_tiling_on_sc=False),
    )(_emb_body)


def kernel(x, token_table, pos_table):
    x_flat = x.reshape(NW, IDX_ROWS, IDX_MINOR)
    out = _build_kernel()(x_flat, token_table, pos_table)
    return out.reshape(B, L, H)
